# baseline (speedup 1.0000x reference)
import functools

import jax
import jax.numpy as jnp
from jax import lax
from jax.experimental import pallas as pl
from jax.experimental.pallas import tpu as pltpu
from jax.experimental.pallas import tpu_sc as plsc

V = 1000000
B = 16384
D = 64
RW = 128
K = 5
NC = 1
NS = 16
NW = NC * NS
BPW = B // NW
C = 64
NCH = BPW // C
L = 16


def _sc_body(pos_v_hbm, pos_u_hbm, negu_hbm, vtab_hbm, utab_hbm,
             spos_hbm, sneg_hbm,
             idxv, idxu, idxn, idxvh, idxuh, idxnh, rv, ru, rn,
             sp, sn, sem):
    wid = lax.axis_index("s") * NC + lax.axis_index("c")
    base = wid * BPW

    def chunk_body(ch, _carry):
        cb = base + ch * C
        pltpu.sync_copy(pos_v_hbm.at[pl.ds(cb, C)], idxv)
        pltpu.sync_copy(pos_u_hbm.at[pl.ds(cb, C)], idxu)
        pltpu.sync_copy(negu_hbm.at[pl.ds(cb * K, C * K)], idxn)
        for i in range(C // L):
            s = pl.ds(i * L, L)
            idxvh[s] = lax.shift_right_logical(idxv[s], 1)
            idxuh[s] = lax.shift_right_logical(idxu[s], 1)
        for i in range(C * K // L):
            s = pl.ds(i * L, L)
            idxnh[s] = lax.shift_right_logical(idxn[s], 1)
        cpv = pltpu.async_copy(vtab_hbm.at[idxvh], rv, sem)
        cpu = pltpu.async_copy(utab_hbm.at[idxuh], ru, sem)
        cpn = pltpu.async_copy(utab_hbm.at[idxnh], rn, sem)
        cpv.wait()
        cpu.wait()
        cpn.wait()

        def group_body(g, _):
            gb = g * L
            hv = (idxv[pl.ds(gb, L)] & 1) * D
            hu = (idxu[pl.ds(gb, L)] & 1) * D
            hn = [(idxn[pl.ds(gb * K + m * L, L)] & 1) * D for m in range(K)]
            for i in range(L):
                b = gb + i
                pv = jnp.zeros((L,), jnp.float32)
                nv = jnp.zeros((L,), jnp.float32)
                for j in range(D // L):
                    vj = rv[b, pl.ds(hv[i] + j * L, L)]
                    uj = ru[b, pl.ds(hu[i] + j * L, L)]
                    pv = pv + vj * uj
                    ii = i * K
                    nsum = rn[b * K, pl.ds(hn[ii // L][ii % L] + j * L, L)]
                    for k in range(1, K):
                        m = ii + k
                        nsum = nsum + rn[b * K + k,
                                         pl.ds(hn[m // L][m % L] + j * L, L)]
                    nv = nv + vj * nsum
                sp[pl.ds((ch * C + b) * L, L)] = pv
                sn[pl.ds((ch * C + b) * L, L)] = nv
            return 0

        lax.fori_loop(0, C // L, group_body, 0)
        return 0

    lax.fori_loop(0, NCH, chunk_body, 0)

    pltpu.sync_copy(sp, spos_hbm.at[pl.ds(base * L, BPW * L)])
    pltpu.sync_copy(sn, sneg_hbm.at[pl.ds(base * L, BPW * L)])


_sc_dots = functools.partial(
    pl.kernel,
    out_type=(jax.ShapeDtypeStruct((B * L,), jnp.float32),
              jax.ShapeDtypeStruct((B * L,), jnp.float32)),
    mesh=plsc.VectorSubcoreMesh(core_axis_name="c", subcore_axis_name="s",
                                num_cores=NC),
    scratch_types=[
        pltpu.VMEM((C,), jnp.int32),
        pltpu.VMEM((C,), jnp.int32),
        pltpu.VMEM((C * K,), jnp.int32),
        pltpu.VMEM((C,), jnp.int32),
        pltpu.VMEM((C,), jnp.int32),
        pltpu.VMEM((C * K,), jnp.int32),
        pltpu.VMEM((C, RW), jnp.float32),
        pltpu.VMEM((C, RW), jnp.float32),
        pltpu.VMEM((C * K, RW), jnp.float32),
        pltpu.VMEM((BPW * L,), jnp.float32),
        pltpu.VMEM((BPW * L,), jnp.float32),
        pltpu.SemaphoreType.DMA,
    ],
)(_sc_body)


def _tc_loss_body(sp_ref, sn_ref, out_ref):
    sp = jnp.sum(sp_ref[...].reshape(B // 8, 8, L), axis=2)
    sn = jnp.sum(sn_ref[...].reshape(B // 8, 8, L), axis=2)

    def logsig(x):
        return jnp.minimum(x, 0.0) - jnp.log1p(jnp.exp(-jnp.abs(x)))

    out_ref[0, 0] = -(jnp.sum(logsig(sp)) + jnp.sum(logsig(-sn)))


_tc_loss = pl.pallas_call(
    _tc_loss_body,
    out_shape=jax.ShapeDtypeStruct((1, 1), jnp.float32),
    out_specs=pl.BlockSpec(memory_space=pltpu.SMEM),
)


def kernel(pos_v, pos_u, neg_u, v_table, u_table):
    pos_v = pos_v.astype(jnp.int32)
    pos_u = pos_u.astype(jnp.int32)
    neg_flat = neg_u.astype(jnp.int32).reshape(-1)
    vt2 = v_table.reshape(V // 2, RW)
    ut2 = u_table.reshape(V // 2, RW)
    sp, sn = _sc_dots(pos_v, pos_u, neg_flat, vt2, ut2)
    loss = _tc_loss(sp.reshape(B // 8, 128), sn.reshape(B // 8, 128))
    return loss[0, 0]


# speedup vs baseline: 1.5763x; 1.5763x over previous
import functools

import jax
import jax.numpy as jnp
from jax import lax
from jax.experimental import pallas as pl
from jax.experimental.pallas import tpu as pltpu
from jax.experimental.pallas import tpu_sc as plsc

V = 1000000
B = 16384
D = 64
K = 5
R = K + 2
NC = 2
NS = 16
NW = NC * NS
BPW = B // NW
C = 64
NCH = BPW // C
L = 16


def _sc_body(pos_v_hbm, pos_u_hbm, negu_hbm, vtab_hbm, utab_hbm,
             spos_hbm, sneg_hbm,
             idxv, idxu, idxn, rows, sp, sn, sem):
    wid = lax.axis_index("s") * NC + lax.axis_index("c")
    base = wid * BPW

    def chunk_body(ch, _carry):
        cb = base + ch * C
        pltpu.sync_copy(pos_v_hbm.at[pl.ds(cb, C)], idxv)
        pltpu.sync_copy(pos_u_hbm.at[pl.ds(cb, C)], idxu)
        pltpu.sync_copy(negu_hbm.at[pl.ds(cb * K, C * K)], idxn)

        def issue(g, _):
            gb = g * L
            vecv = idxv[pl.ds(gb, L)]
            vecu = idxu[pl.ds(gb, L)]
            vecn = [idxn[pl.ds(gb * K + k * L, L)] for k in range(K)]
            for i in range(L):
                rb = (gb + i) * R
                pltpu.async_copy(vtab_hbm.at[pl.ds(vecv[i], 1)],
                                 rows.at[pl.ds(rb, 1)], sem)
                pltpu.async_copy(utab_hbm.at[pl.ds(vecu[i], 1)],
                                 rows.at[pl.ds(rb + 1, 1)], sem)
                ii = i * K
                for k in range(K):
                    jn = vecn[(ii + k) // L][(ii + k) % L]
                    pltpu.async_copy(utab_hbm.at[pl.ds(jn, 1)],
                                     rows.at[pl.ds(rb + 2 + k, 1)], sem)
            return 0

        lax.fori_loop(0, C // L, issue, 0)
        pltpu.make_async_copy(vtab_hbm.at[pl.ds(0, C * R)], rows, sem).wait()

        def compute_b(b, _):
            rb = b * R
            pv = jnp.zeros((L,), jnp.float32)
            nv = jnp.zeros((L,), jnp.float32)
            for j in range(D // L):
                s = pl.ds(j * L, L)
                vj = rows[rb, s]
                pv = pv + vj * rows[rb + 1, s]
                nsum = rows[rb + 2, s]
                for k in range(1, K):
                    nsum = nsum + rows[rb + 2 + k, s]
                nv = nv + vj * nsum
            sp[pl.ds((ch * C + b) * L, L)] = pv
            sn[pl.ds((ch * C + b) * L, L)] = nv
            return 0

        lax.fori_loop(0, C, compute_b, 0)
        return 0

    lax.fori_loop(0, NCH, chunk_body, 0)

    pltpu.sync_copy(sp, spos_hbm.at[pl.ds(base * L, BPW * L)])
    pltpu.sync_copy(sn, sneg_hbm.at[pl.ds(base * L, BPW * L)])


_sc_dots = functools.partial(
    pl.kernel,
    out_type=(jax.ShapeDtypeStruct((B * L,), jnp.float32),
              jax.ShapeDtypeStruct((B * L,), jnp.float32)),
    mesh=plsc.VectorSubcoreMesh(core_axis_name="c", subcore_axis_name="s",
                                num_cores=NC),
    scratch_types=[
        pltpu.VMEM((C,), jnp.int32),
        pltpu.VMEM((C,), jnp.int32),
        pltpu.VMEM((C * K,), jnp.int32),
        pltpu.VMEM((C * R, D), jnp.float32),
        pltpu.VMEM((BPW * L,), jnp.float32),
        pltpu.VMEM((BPW * L,), jnp.float32),
        pltpu.SemaphoreType.DMA,
    ],
)(_sc_body)


def _tc_loss_body(sp_ref, sn_ref, out_ref):
    sp = jnp.sum(sp_ref[...].reshape(B // 8, 8, L), axis=2)
    sn = jnp.sum(sn_ref[...].reshape(B // 8, 8, L), axis=2)

    def logsig(x):
        return jnp.minimum(x, 0.0) - jnp.log1p(jnp.exp(-jnp.abs(x)))

    out_ref[0, 0] = -(jnp.sum(logsig(sp)) + jnp.sum(logsig(-sn)))


_tc_loss = pl.pallas_call(
    _tc_loss_body,
    out_shape=jax.ShapeDtypeStruct((1, 1), jnp.float32),
    out_specs=pl.BlockSpec(memory_space=pltpu.SMEM),
)


def kernel(pos_v, pos_u, neg_u, v_table, u_table):
    pos_v = pos_v.astype(jnp.int32)
    pos_u = pos_u.astype(jnp.int32)
    neg_flat = neg_u.astype(jnp.int32).reshape(-1)
    sp, sn = _sc_dots(pos_v, pos_u, neg_flat, v_table, u_table)
    loss = _tc_loss(sp.reshape(B // 8, 128), sn.reshape(B // 8, 128))
    return loss[0, 0]
